# baseline (speedup 1.0000x reference)
import functools

import jax
import jax.numpy as jnp
from jax import lax
from jax.experimental import pallas as pl
from jax.experimental.pallas import tpu as pltpu
from jax.experimental.pallas import tpu_sc as plsc

_T = 512
_R = 128
_DST = _T - _R

_NCORE = 2
_NSUB = 16
_TOK_TILE = _T // _NSUB
_ROW_TILE = _DST // _NSUB


def _match_body(metric_ref, q_ref, ts_c_ref, ts_r_ref, coef_ref, targ_ref):
    f32 = jnp.float32
    mn = metric_ref[0]
    qn = q_ref[0]
    ts_c = ts_c_ref[0]
    ts_r = ts_r_ref[0]
    NEG = jnp.float32(-jnp.inf)

    sim = lax.dot_general(mn, mn, (((1,), (1,)), ((), ())),
                          preferred_element_type=f32)
    ri = lax.broadcasted_iota(jnp.int32, (_T, _T), 0)
    ci = lax.broadcasted_iota(jnp.int32, (_T, _T), 1)
    bad = (ri == ci) | (ri == 0) | (ci == 0) | (ri == _T - 1) | (ci == _T - 1)
    simm = jnp.where(bad, NEG, sim)

    rm_c = jnp.max(simm, axis=1, keepdims=True)
    rm_r = jnp.max(simm, axis=0, keepdims=True)

    vm_c = (rm_r < rm_c) | ((rm_r == rm_c) & (ci > ri))
    cmax_c = jnp.max(jnp.where(vm_c, simm, NEG), axis=1, keepdims=True)
    vm_r = (rm_c < rm_r) | ((rm_c == rm_r) & (ri > ci))
    cmax_r = jnp.max(jnp.where(vm_r, simm, NEG), axis=0, keepdims=True)

    imp_c = lax.dot_general(mn.astype(jnp.bfloat16),
                            jnp.broadcast_to(qn, (8, qn.shape[1])
                                             ).astype(jnp.bfloat16),
                            (((1,), (1,)), ((), ())),
                            preferred_element_type=f32)[:, 0:1]
    imp_r = jnp.sum(jnp.where(ri == ci, imp_c, 0.0), axis=0,
                    keepdims=True)
    score_c = imp_c - cmax_c
    score_r = imp_r - cmax_r

    lt_c = (score_r < score_c) | ((score_r == score_c) & (ci < ri))
    srank_c = jnp.sum(lt_c.astype(jnp.int32), axis=1, keepdims=True)
    lt_r = (score_c < score_r) | ((score_c == score_r) & (ri < ci))
    srank_r = jnp.sum(lt_r.astype(jnp.int32), axis=0, keepdims=True)

    issrc_c = srank_c < _R
    issrc_r = srank_r < _R
    isdst_c = ~issrc_c
    isdst_r = ~issrc_r

    posdst_r = jnp.sum((isdst_c & (ri < ci)).astype(jnp.int32), axis=0,
                       keepdims=True)
    posdst_c = jnp.sum((isdst_r & (ci < ri)).astype(jnp.int32), axis=1,
                       keepdims=True)

    BIG = jnp.int32(1 << 20)
    mx_r = jnp.max(jnp.where(isdst_c, simm, NEG), axis=0, keepdims=True)
    cand_r = isdst_c & (simm == mx_r)
    dsr_r = jnp.min(jnp.where(cand_r, srank_c, BIG), axis=0, keepdims=True)
    hit_r = srank_c == dsr_r
    impd_r = jnp.sum(jnp.where(hit_r, imp_c, 0.0), axis=0, keepdims=True)
    posd_r = jnp.sum(jnp.where(hit_r, posdst_c, 0), axis=0, keepdims=True)
    tsd_r = jnp.sum(jnp.where(hit_r, ts_c, 0.0), axis=0, keepdims=True)

    mx_c = jnp.max(jnp.where(isdst_r, simm, NEG), axis=1, keepdims=True)
    cand_c = isdst_r & (simm == mx_c)
    dsr_c = jnp.min(jnp.where(cand_c, srank_r, BIG), axis=1, keepdims=True)
    hit_c = srank_r == dsr_c
    posd_c = jnp.sum(jnp.where(hit_c, posdst_r, 0), axis=1, keepdims=True)

    b_r = 2.0 / (1.0 + jnp.exp(impd_r - imp_r))
    impd_c = jnp.sum(jnp.where(hit_c, imp_r, 0.0), axis=1, keepdims=True)
    b_c2 = 2.0 / (1.0 + jnp.exp(impd_c - imp_c))

    tgt_match = issrc_c & (posd_c == posdst_r)
    cntd_r = jnp.sum(tgt_match.astype(f32), axis=0, keepdims=True)
    Bd_r = jnp.sum(jnp.where(tgt_match, b_c2, 0.0), axis=0, keepdims=True)
    tsden_r = ts_r + jnp.sum(jnp.where(tgt_match, ts_c, 0.0), axis=0,
                             keepdims=True)
    src_share = issrc_c & (posd_c == posd_r)
    den_src_r = tsd_r + jnp.sum(jnp.where(src_share, ts_c, 0.0), axis=0,
                                keepdims=True)

    coefA_r = b_r / den_src_r
    coefB_r = (1.0 + cntd_r - Bd_r) / tsden_r

    coef_ref[0] = jnp.where(issrc_r, coefA_r, coefB_r) * ts_r
    targ_ref[0] = jnp.where(issrc_r, posd_r, posdst_r)


def _match(mn, qn, ts_c, ts_r):
    n, t, cm = mn.shape
    return pl.pallas_call(
        _match_body,
        grid=(n,),
        in_specs=[
            pl.BlockSpec((1, t, cm), lambda i: (i, 0, 0)),
            pl.BlockSpec((1, 1, cm), lambda i: (i, 0, 0)),
            pl.BlockSpec((1, t, 1), lambda i: (i, 0, 0)),
            pl.BlockSpec((1, 1, t), lambda i: (i, 0, 0)),
        ],
        out_specs=[
            pl.BlockSpec((1, 1, t), lambda i: (i, 0, 0)),
            pl.BlockSpec((1, 1, t), lambda i: (i, 0, 0)),
        ],
        out_shape=[
            jax.ShapeDtypeStruct((n, 1, t), jnp.float32),
            jax.ShapeDtypeStruct((n, 1, t), jnp.int32),
        ],
    )(mn, qn, ts_c, ts_r)


def _make_merge(n, c):
    samp_per_core = n // _NCORE
    mesh = plsc.VectorSubcoreMesh(core_axis_name="c", subcore_axis_name="s")

    @functools.partial(
        pl.kernel,
        mesh=mesh,
        compiler_params=pltpu.CompilerParams(use_tc_tiling_on_sc=False,
                                             needs_layout_passes=False),
        out_type=jax.ShapeDtypeStruct((n * _DST, c), jnp.float32),
        scratch_types=[
            pltpu.VMEM((2, _TOK_TILE, c), jnp.float32),
            pltpu.VMEM((2, _TOK_TILE), jnp.float32),
            pltpu.VMEM((2, _TOK_TILE), jnp.int32),
            pltpu.VMEM((8, c), jnp.float32),
            pltpu.VMEM_SHARED((2, _DST, c), jnp.float32),
            pltpu.SemaphoreType.DMA,
            pltpu.SemaphoreType.DMA,
            pltpu.SemaphoreType.DMA,
            pltpu.SemaphoreType.DMA,
            pltpu.SemaphoreType.DMA,
        ],
    )
    def merge(x_hbm, coef_hbm, targ_hbm, zeros_hbm, out_hbm,
              rows_v, coef_v, targ_v, zero_v, accum,
              semx, semc, semt, semz, semo):
        cid = lax.axis_index("c")
        sid = lax.axis_index("s")
        pltpu.sync_copy(zeros_hbm, zero_v)
        lanes = lax.iota(jnp.int32, 16)

        def issue_loads(s, buf):
            s = jnp.minimum(s, samp_per_core - 1)
            nsamp = cid * samp_per_core + s
            tokbase = nsamp * _T + sid * _TOK_TILE
            pltpu.async_copy(coef_hbm.at[pl.ds(tokbase, _TOK_TILE)],
                             coef_v.at[buf], semc)
            pltpu.async_copy(targ_hbm.at[pl.ds(tokbase, _TOK_TILE)],
                             targ_v.at[buf], semt)
            pltpu.async_copy(x_hbm.at[pl.ds(tokbase, _TOK_TILE)],
                             rows_v.at[buf], semx)

        def drain_loads(buf):
            pltpu.make_async_copy(coef_hbm.at[pl.ds(0, _TOK_TILE)],
                                  coef_v.at[buf], semc).wait()
            pltpu.make_async_copy(targ_hbm.at[pl.ds(0, _TOK_TILE)],
                                  targ_v.at[buf], semt).wait()
            pltpu.make_async_copy(x_hbm.at[pl.ds(0, _TOK_TILE)],
                                  rows_v.at[buf], semx).wait()

        def issue_zeros(a):
            for i in range(_ROW_TILE // 8):
                pltpu.async_copy(
                    zero_v,
                    accum.at[a, pl.ds(sid * _ROW_TILE + i * 8, 8)], semz)

        def drain_zeros(a):
            for i in range(_ROW_TILE // 8):
                pltpu.make_async_copy(
                    zero_v,
                    accum.at[a, pl.ds(sid * _ROW_TILE + i * 8, 8)],
                    semz).wait()

        def scale(buf):
            ck = coef_v.at[buf]

            def tok(t2, carry2):
                cval = plsc.load_gather(ck, [lanes * 0 + t2])
                row = rows_v.at[buf, t2]
                for j in range(c // 16):
                    row[pl.ds(j * 16, 16)] = row[pl.ds(j * 16, 16)] * cval
                return carry2

            lax.fori_loop(0, _TOK_TILE, tok, 0)

        def out_sample(s, a):
            nsamp = cid * samp_per_core + s
            outbase = nsamp * _DST + sid * _ROW_TILE
            pltpu.sync_copy(
                accum.at[a, pl.ds(sid * _ROW_TILE, _ROW_TILE)],
                out_hbm.at[pl.ds(outbase, _ROW_TILE)])
            issue_zeros(a)

        def phase(s, buf, emit_prev):
            issue_loads(s + 1, 1 - buf)
            drain_loads(buf)
            scale(buf)
            drain_zeros(buf)
            plsc.subcore_barrier()
            if emit_prev:
                out_sample(s - 1, 1 - buf)
            pltpu.sync_copy(rows_v.at[buf], accum.at[buf].at[targ_v.at[buf]],
                            add=True)

        issue_loads(jnp.int32(0), 0)
        issue_zeros(0)
        issue_zeros(1)

        phase(jnp.int32(0), 0, False)

        def body(rr, carry):
            s0 = rr * 2 + 1
            phase(s0, 1, True)
            phase(s0 + 1, 0, True)
            return carry

        lax.fori_loop(0, (samp_per_core - 2) // 2, body, 0)
        phase(jnp.int32(samp_per_core - 1), 1, True)
        plsc.subcore_barrier()
        out_sample(samp_per_core - 1, 1)
        drain_loads(0)
        drain_zeros(0)
        drain_zeros(1)

    return merge


def kernel(x, query, metric, token_size, r):
    del r
    n, t, c = x.shape
    qlast = query[:, -1:, :]
    mn = metric / jnp.linalg.norm(metric, axis=-1, keepdims=True)
    qn = qlast / jnp.linalg.norm(qlast, axis=-1, keepdims=True)
    ts_c = token_size
    ts_r = jnp.swapaxes(token_size, 1, 2)

    zeros = jnp.zeros((8, c), jnp.float32)
    coef, targ = _match(mn, qn, ts_c, ts_r)
    xflat = x.reshape(n * t, c)
    out = _make_merge(n, c)(xflat, coef.reshape(n * t),
                            targ.reshape(n * t), zeros)
    return out.reshape(n, _DST, c)


# speedup vs baseline: 1.0342x; 1.0342x over previous
import functools

import jax
import jax.numpy as jnp
from jax import lax
from jax.experimental import pallas as pl
from jax.experimental.pallas import tpu as pltpu
from jax.experimental.pallas import tpu_sc as plsc

_T = 512
_R = 128
_DST = _T - _R

_NCORE = 2
_NSUB = 16
_TOK_TILE = _T // _NSUB
_ROW_TILE = _DST // _NSUB


def _match_body(metric_ref, q_ref, ts_c_ref, ts_r_ref, coef_ref, targ_ref):
    f32 = jnp.float32
    mn = metric_ref[0]
    qn = q_ref[0]
    ts_c = ts_c_ref[0]
    ts_r = ts_r_ref[0]
    NEG = jnp.float32(-jnp.inf)

    sim = lax.dot_general(mn, mn, (((1,), (1,)), ((), ())),
                          preferred_element_type=f32)
    ri = lax.broadcasted_iota(jnp.int32, (_T, _T), 0)
    ci = lax.broadcasted_iota(jnp.int32, (_T, _T), 1)
    bad = (ri == ci) | (ri == 0) | (ci == 0) | (ri == _T - 1) | (ci == _T - 1)
    simm = jnp.where(bad, NEG, sim)

    rm_c = jnp.max(simm, axis=1, keepdims=True)
    rm_r = jnp.max(simm, axis=0, keepdims=True)

    vm_c = (rm_r < rm_c) | ((rm_r == rm_c) & (ci > ri))
    cmax_c = jnp.max(jnp.where(vm_c, simm, NEG), axis=1, keepdims=True)
    vm_r = (rm_c < rm_r) | ((rm_c == rm_r) & (ri > ci))
    cmax_r = jnp.max(jnp.where(vm_r, simm, NEG), axis=0, keepdims=True)

    imp_c = lax.dot_general(mn.astype(jnp.bfloat16),
                            jnp.broadcast_to(qn, (8, qn.shape[1])
                                             ).astype(jnp.bfloat16),
                            (((1,), (1,)), ((), ())),
                            preferred_element_type=f32)[:, 0:1]
    imp_r = jnp.sum(jnp.where(ri == ci, imp_c, 0.0), axis=0,
                    keepdims=True)
    score_c = imp_c - cmax_c
    score_r = imp_r - cmax_r

    lt_c = (score_r < score_c) | ((score_r == score_c) & (ci < ri))
    srank_c = jnp.sum(lt_c.astype(jnp.int32), axis=1, keepdims=True)
    lt_r = (score_c < score_r) | ((score_c == score_r) & (ri < ci))
    srank_r = jnp.sum(lt_r.astype(jnp.int32), axis=0, keepdims=True)

    issrc_c = srank_c < _R
    issrc_r = srank_r < _R
    isdst_c = ~issrc_c
    isdst_r = ~issrc_r

    posdst_r = jnp.sum((isdst_c & (ri < ci)).astype(jnp.int32), axis=0,
                       keepdims=True)
    posdst_c = jnp.sum((isdst_r & (ci < ri)).astype(jnp.int32), axis=1,
                       keepdims=True)

    BIG = jnp.int32(1 << 20)
    mx_r = jnp.max(jnp.where(isdst_c, simm, NEG), axis=0, keepdims=True)
    cand_r = isdst_c & (simm == mx_r)
    dsr_r = jnp.min(jnp.where(cand_r, srank_c, BIG), axis=0, keepdims=True)
    hit_r = srank_c == dsr_r
    impd_r = jnp.sum(jnp.where(hit_r, imp_c, 0.0), axis=0, keepdims=True)
    posd_r = jnp.sum(jnp.where(hit_r, posdst_c, 0), axis=0, keepdims=True)
    tsd_r = jnp.sum(jnp.where(hit_r, ts_c, 0.0), axis=0, keepdims=True)

    mx_c = jnp.max(jnp.where(isdst_r, simm, NEG), axis=1, keepdims=True)
    cand_c = isdst_r & (simm == mx_c)
    dsr_c = jnp.min(jnp.where(cand_c, srank_r, BIG), axis=1, keepdims=True)
    hit_c = srank_r == dsr_c
    posd_c = jnp.sum(jnp.where(hit_c, posdst_r, 0), axis=1, keepdims=True)

    b_r = 2.0 / (1.0 + jnp.exp(impd_r - imp_r))
    impd_c = jnp.sum(jnp.where(hit_c, imp_r, 0.0), axis=1, keepdims=True)
    b_c2 = 2.0 / (1.0 + jnp.exp(impd_c - imp_c))

    tgt_match = issrc_c & (posd_c == posdst_r)
    cntd_r = jnp.sum(tgt_match.astype(f32), axis=0, keepdims=True)
    Bd_r = jnp.sum(jnp.where(tgt_match, b_c2, 0.0), axis=0, keepdims=True)
    tsden_r = ts_r + jnp.sum(jnp.where(tgt_match, ts_c, 0.0), axis=0,
                             keepdims=True)
    src_share = issrc_c & (posd_c == posd_r)
    den_src_r = tsd_r + jnp.sum(jnp.where(src_share, ts_c, 0.0), axis=0,
                                keepdims=True)

    coefA_r = b_r / den_src_r
    coefB_r = (1.0 + cntd_r - Bd_r) / tsden_r

    coef_ref[0] = jnp.where(issrc_r, coefA_r, coefB_r) * ts_r
    targ_ref[0] = jnp.where(issrc_r, posd_r, posdst_r)


def _match(mn, qn, ts_c, ts_r):
    n, t, cm = mn.shape
    return pl.pallas_call(
        _match_body,
        grid=(n,),
        in_specs=[
            pl.BlockSpec((1, t, cm), lambda i: (i, 0, 0)),
            pl.BlockSpec((1, 1, cm), lambda i: (i, 0, 0)),
            pl.BlockSpec((1, t, 1), lambda i: (i, 0, 0)),
            pl.BlockSpec((1, 1, t), lambda i: (i, 0, 0)),
        ],
        out_specs=[
            pl.BlockSpec((1, 1, t), lambda i: (i, 0, 0)),
            pl.BlockSpec((1, 1, t), lambda i: (i, 0, 0)),
        ],
        out_shape=[
            jax.ShapeDtypeStruct((n, 1, t), jnp.float32),
            jax.ShapeDtypeStruct((n, 1, t), jnp.int32),
        ],
    )(mn, qn, ts_c, ts_r)


def _make_merge(n, c):
    samp_per_core = n // _NCORE
    mesh = plsc.VectorSubcoreMesh(core_axis_name="c", subcore_axis_name="s")

    @functools.partial(
        pl.kernel,
        mesh=mesh,
        compiler_params=pltpu.CompilerParams(use_tc_tiling_on_sc=False,
                                             needs_layout_passes=False),
        out_type=jax.ShapeDtypeStruct((n * _DST, c), jnp.float32),
        scratch_types=[
            pltpu.VMEM((2, _TOK_TILE, c), jnp.float32),
            pltpu.VMEM((2, _TOK_TILE), jnp.float32),
            pltpu.VMEM((2, _TOK_TILE), jnp.int32),
            pltpu.VMEM((8, c), jnp.float32),
            pltpu.VMEM_SHARED((2, _DST, c), jnp.float32),
            pltpu.SemaphoreType.DMA,
            pltpu.SemaphoreType.DMA,
            pltpu.SemaphoreType.DMA,
            pltpu.SemaphoreType.DMA,
            pltpu.SemaphoreType.DMA,
        ],
    )
    def merge(x_hbm, coef_hbm, targ_hbm, zeros_hbm, out_hbm,
              rows_v, coef_v, targ_v, zero_v, accum,
              semx, semc, semt, semz, semo):
        cid = lax.axis_index("c")
        sid = lax.axis_index("s")
        pltpu.sync_copy(zeros_hbm, zero_v)
        lanes = lax.iota(jnp.int32, 16)

        def issue_loads(s, buf):
            s = jnp.minimum(s, samp_per_core - 1)
            nsamp = cid * samp_per_core + s
            tokbase = nsamp * _T + sid * _TOK_TILE
            pltpu.async_copy(coef_hbm.at[pl.ds(tokbase, _TOK_TILE)],
                             coef_v.at[buf], semc)
            pltpu.async_copy(targ_hbm.at[pl.ds(tokbase, _TOK_TILE)],
                             targ_v.at[buf], semt)
            pltpu.async_copy(x_hbm.at[pl.ds(tokbase, _TOK_TILE)],
                             rows_v.at[buf], semx)

        def drain_loads(buf):
            pltpu.make_async_copy(coef_hbm.at[pl.ds(0, _TOK_TILE)],
                                  coef_v.at[buf], semc).wait()
            pltpu.make_async_copy(targ_hbm.at[pl.ds(0, _TOK_TILE)],
                                  targ_v.at[buf], semt).wait()
            pltpu.make_async_copy(x_hbm.at[pl.ds(0, _TOK_TILE)],
                                  rows_v.at[buf], semx).wait()

        def issue_zeros(a):
            for i in range(_ROW_TILE // 8):
                pltpu.async_copy(
                    zero_v,
                    accum.at[a, pl.ds(sid * _ROW_TILE + i * 8, 8)], semz)

        def drain_zeros(a):
            for i in range(_ROW_TILE // 8):
                pltpu.make_async_copy(
                    zero_v,
                    accum.at[a, pl.ds(sid * _ROW_TILE + i * 8, 8)],
                    semz).wait()

        def process(s, buf):
            ck = coef_v.at[buf]

            def tok(t2, carry2):
                cval = plsc.load_gather(ck, [lanes * 0 + t2])
                row = rows_v.at[buf, t2]
                for j in range(c // 16):
                    row[pl.ds(j * 16, 16)] = row[pl.ds(j * 16, 16)] * cval
                return carry2

            lax.fori_loop(0, _TOK_TILE, tok, 0)
            drain_zeros(buf)
            plsc.subcore_barrier()
            pltpu.sync_copy(rows_v.at[buf], accum.at[buf].at[targ_v.at[buf]],
                            add=True)
            plsc.subcore_barrier()
            nsamp = cid * samp_per_core + s
            outbase = nsamp * _DST + sid * _ROW_TILE
            pltpu.sync_copy(
                accum.at[buf, pl.ds(sid * _ROW_TILE, _ROW_TILE)],
                out_hbm.at[pl.ds(outbase, _ROW_TILE)])
            issue_zeros(buf)

        issue_loads(jnp.int32(0), 0)
        issue_zeros(0)
        issue_zeros(1)

        def body(rr, carry):
            s0 = rr * 2
            issue_loads(s0 + 1, 1)
            drain_loads(0)
            process(s0, 0)
            issue_loads(s0 + 2, 0)
            drain_loads(1)
            process(s0 + 1, 1)
            return carry

        lax.fori_loop(0, samp_per_core // 2, body, 0)
        drain_loads(0)
        drain_zeros(0)
        drain_zeros(1)

    return merge


def kernel(x, query, metric, token_size, r):
    del r
    n, t, c = x.shape
    qlast = query[:, -1:, :]
    mn = metric / jnp.linalg.norm(metric, axis=-1, keepdims=True)
    qn = qlast / jnp.linalg.norm(qlast, axis=-1, keepdims=True)
    ts_c = token_size
    ts_r = jnp.swapaxes(token_size, 1, 2)

    zeros = jnp.zeros((8, c), jnp.float32)
    coef, targ = _match(mn, qn, ts_c, ts_r)
    xflat = x.reshape(n * t, c)
    out = _make_merge(n, c)(xflat, coef.reshape(n * t),
                            targ.reshape(n * t), zeros)
    return out.reshape(n, _DST, c)


# speedup vs baseline: 1.0896x; 1.0536x over previous
import functools

import jax
import jax.numpy as jnp
from jax import lax
from jax.experimental import pallas as pl
from jax.experimental.pallas import tpu as pltpu
from jax.experimental.pallas import tpu_sc as plsc

_T = 512
_R = 128
_DST = _T - _R

_NCORE = 2
_NSUB = 16
_TOK_TILE = _T // _NSUB
_ROW_TILE = _DST // _NSUB


def _match_body(metric_ref, q_ref, ts_c_ref, ts_r_ref, coef_ref, targ_ref):
    f32 = jnp.float32
    mn = metric_ref[0]
    qn = q_ref[0]
    ts_c = ts_c_ref[0]
    ts_r = ts_r_ref[0]
    NEG = jnp.float32(-jnp.inf)

    sim = lax.dot_general(mn, mn, (((1,), (1,)), ((), ())),
                          preferred_element_type=f32)
    ri = lax.broadcasted_iota(jnp.int32, (_T, _T), 0)
    ci = lax.broadcasted_iota(jnp.int32, (_T, _T), 1)
    bad = (ri == ci) | (ri == 0) | (ci == 0) | (ri == _T - 1) | (ci == _T - 1)
    simm = jnp.where(bad, NEG, sim)

    rm_c = jnp.max(simm, axis=1, keepdims=True)
    rm_r = jnp.max(simm, axis=0, keepdims=True)

    vm_c = (rm_r < rm_c) | ((rm_r == rm_c) & (ci > ri))
    cmax_c = jnp.max(jnp.where(vm_c, simm, NEG), axis=1, keepdims=True)
    vm_r = (rm_c < rm_r) | ((rm_c == rm_r) & (ri > ci))
    cmax_r = jnp.max(jnp.where(vm_r, simm, NEG), axis=0, keepdims=True)

    imp_c = lax.dot_general(mn.astype(jnp.bfloat16),
                            jnp.broadcast_to(qn, (8, qn.shape[1])
                                             ).astype(jnp.bfloat16),
                            (((1,), (1,)), ((), ())),
                            preferred_element_type=f32)[:, 0:1]
    imp_r = jnp.sum(jnp.where(ri == ci, imp_c, 0.0), axis=0,
                    keepdims=True)
    score_c = imp_c - cmax_c
    score_r = imp_r - cmax_r

    lt_c = (score_r < score_c) | ((score_r == score_c) & (ci < ri))
    srank_c = jnp.sum(lt_c.astype(jnp.int32), axis=1, keepdims=True)
    lt_r = (score_c < score_r) | ((score_c == score_r) & (ri < ci))
    srank_r = jnp.sum(lt_r.astype(jnp.int32), axis=0, keepdims=True)

    issrc_c = srank_c < _R
    issrc_r = srank_r < _R
    isdst_c = ~issrc_c
    isdst_r = ~issrc_r

    posdst_r = jnp.sum((isdst_c & (ri < ci)).astype(jnp.int32), axis=0,
                       keepdims=True)
    posdst_c = jnp.sum((isdst_r & (ci < ri)).astype(jnp.int32), axis=1,
                       keepdims=True)

    BIG = jnp.int32(1 << 20)
    mx_r = jnp.max(jnp.where(isdst_c, simm, NEG), axis=0, keepdims=True)
    cand_r = isdst_c & (simm == mx_r)
    dsr_r = jnp.min(jnp.where(cand_r, srank_c, BIG), axis=0, keepdims=True)
    hit_r = srank_c == dsr_r
    impd_r = jnp.sum(jnp.where(hit_r, imp_c, 0.0), axis=0, keepdims=True)
    posd_r = jnp.sum(jnp.where(hit_r, posdst_c, 0), axis=0, keepdims=True)
    tsd_r = jnp.sum(jnp.where(hit_r, ts_c, 0.0), axis=0, keepdims=True)

    mx_c = jnp.max(jnp.where(isdst_r, simm, NEG), axis=1, keepdims=True)
    cand_c = isdst_r & (simm == mx_c)
    dsr_c = jnp.min(jnp.where(cand_c, srank_r, BIG), axis=1, keepdims=True)
    hit_c = srank_r == dsr_c
    posd_c = jnp.sum(jnp.where(hit_c, posdst_r, 0), axis=1, keepdims=True)

    b_r = 2.0 / (1.0 + jnp.exp(impd_r - imp_r))
    impd_c = jnp.sum(jnp.where(hit_c, imp_r, 0.0), axis=1, keepdims=True)
    b_c2 = 2.0 / (1.0 + jnp.exp(impd_c - imp_c))

    tgt_match = issrc_c & (posd_c == posdst_r)
    cntd_r = jnp.sum(tgt_match.astype(f32), axis=0, keepdims=True)
    Bd_r = jnp.sum(jnp.where(tgt_match, b_c2, 0.0), axis=0, keepdims=True)
    tsden_r = ts_r + jnp.sum(jnp.where(tgt_match, ts_c, 0.0), axis=0,
                             keepdims=True)
    src_share = issrc_c & (posd_c == posd_r)
    den_src_r = tsd_r + jnp.sum(jnp.where(src_share, ts_c, 0.0), axis=0,
                                keepdims=True)

    coefA_r = b_r / den_src_r
    coefB_r = (1.0 + cntd_r - Bd_r) / tsden_r

    coef_ref[0] = jnp.where(issrc_r, coefA_r, coefB_r) * ts_r
    targ_ref[0] = jnp.where(issrc_r, posd_r, posdst_r)


def _match(mn, qn, ts_c, ts_r):
    n, t, cm = mn.shape
    return pl.pallas_call(
        _match_body,
        grid=(n,),
        in_specs=[
            pl.BlockSpec((1, t, cm), lambda i: (i, 0, 0)),
            pl.BlockSpec((1, 1, cm), lambda i: (i, 0, 0)),
            pl.BlockSpec((1, t, 1), lambda i: (i, 0, 0)),
            pl.BlockSpec((1, 1, t), lambda i: (i, 0, 0)),
        ],
        out_specs=[
            pl.BlockSpec((1, 1, t), lambda i: (i, 0, 0)),
            pl.BlockSpec((1, 1, t), lambda i: (i, 0, 0)),
        ],
        out_shape=[
            jax.ShapeDtypeStruct((n, 1, t), jnp.float32),
            jax.ShapeDtypeStruct((n, 1, t), jnp.int32),
        ],
    )(mn, qn, ts_c, ts_r)


def _make_merge(n, c):
    samp_per_core = n // _NCORE
    mesh = plsc.VectorSubcoreMesh(core_axis_name="c", subcore_axis_name="s")

    @functools.partial(
        pl.kernel,
        mesh=mesh,
        compiler_params=pltpu.CompilerParams(use_tc_tiling_on_sc=False,
                                             needs_layout_passes=False),
        out_type=jax.ShapeDtypeStruct((n * _DST, c), jnp.float32),
        scratch_types=[
            pltpu.VMEM((2, _TOK_TILE, c), jnp.float32),
            pltpu.VMEM((2, _TOK_TILE), jnp.float32),
            pltpu.VMEM((2, _TOK_TILE), jnp.int32),
            pltpu.VMEM((8, c), jnp.float32),
            pltpu.VMEM_SHARED((2, _DST, c), jnp.float32),
            pltpu.SemaphoreType.DMA,
            pltpu.SemaphoreType.DMA,
            pltpu.SemaphoreType.DMA,
            pltpu.SemaphoreType.DMA,
            pltpu.SemaphoreType.DMA,
        ],
    )
    def merge(x_hbm, coef_hbm, targ_hbm, zeros_hbm, out_hbm,
              rows_v, coef_v, targ_v, zero_v, accum,
              semx, semc, semt, semz, semo):
        cid = lax.axis_index("c")
        sid = lax.axis_index("s")
        pltpu.sync_copy(zeros_hbm, zero_v)
        lanes = lax.iota(jnp.int32, 16)

        def issue_loads(s, buf):
            s = jnp.minimum(s, samp_per_core - 1)
            nsamp = cid * samp_per_core + s
            tokbase = nsamp * _T + sid * _TOK_TILE
            pltpu.async_copy(coef_hbm.at[pl.ds(tokbase, _TOK_TILE)],
                             coef_v.at[buf], semc)
            pltpu.async_copy(targ_hbm.at[pl.ds(tokbase, _TOK_TILE)],
                             targ_v.at[buf], semt)
            pltpu.async_copy(x_hbm.at[pl.ds(tokbase, _TOK_TILE)],
                             rows_v.at[buf], semx)

        def drain_loads(buf):
            pltpu.make_async_copy(coef_hbm.at[pl.ds(0, _TOK_TILE)],
                                  coef_v.at[buf], semc).wait()
            pltpu.make_async_copy(targ_hbm.at[pl.ds(0, _TOK_TILE)],
                                  targ_v.at[buf], semt).wait()
            pltpu.make_async_copy(x_hbm.at[pl.ds(0, _TOK_TILE)],
                                  rows_v.at[buf], semx).wait()

        def issue_zeros(a):
            for i in range(_ROW_TILE // 8):
                pltpu.async_copy(
                    zero_v,
                    accum.at[a, pl.ds(sid * _ROW_TILE + i * 8, 8)], semz)

        def drain_zeros(a):
            for i in range(_ROW_TILE // 8):
                pltpu.make_async_copy(
                    zero_v,
                    accum.at[a, pl.ds(sid * _ROW_TILE + i * 8, 8)],
                    semz).wait()

        def scale(buf):
            ck = coef_v.at[buf]

            def tok(t2, carry2):
                cval = plsc.load_gather(ck, [lanes * 0 + t2])
                row = rows_v.at[buf, t2]
                for j in range(c // 16):
                    row[pl.ds(j * 16, 16)] = row[pl.ds(j * 16, 16)] * cval
                return carry2

            lax.fori_loop(0, _TOK_TILE, tok, 0)

        def issue_out(s, buf):
            nsamp = cid * samp_per_core + s
            outbase = nsamp * _DST + sid * _ROW_TILE
            pltpu.async_copy(
                accum.at[buf, pl.ds(sid * _ROW_TILE, _ROW_TILE)],
                out_hbm.at[pl.ds(outbase, _ROW_TILE)], semo)

        def drain_out(buf):
            pltpu.make_async_copy(
                accum.at[buf, pl.ds(sid * _ROW_TILE, _ROW_TILE)],
                out_hbm.at[pl.ds(0, _ROW_TILE)], semo).wait()

        def phase(s, buf, first):
            issue_loads(s + 1, 1 - buf)
            drain_loads(buf)
            scale(buf)
            if not first:
                drain_out(1 - buf)
                issue_zeros(1 - buf)
            drain_zeros(buf)
            plsc.subcore_barrier()
            pltpu.sync_copy(rows_v.at[buf], accum.at[buf].at[targ_v.at[buf]],
                            add=True)
            plsc.subcore_barrier()
            issue_out(s, buf)

        issue_loads(jnp.int32(0), 0)
        issue_zeros(0)
        issue_zeros(1)

        phase(jnp.int32(0), 0, True)

        def body(rr, carry):
            s0 = rr * 2 + 1
            phase(s0, 1, False)
            phase(s0 + 1, 0, False)
            return carry

        lax.fori_loop(0, (samp_per_core - 2) // 2, body, 0)
        phase(jnp.int32(samp_per_core - 1), 1, False)
        drain_out(1)
        drain_zeros(0)
        drain_loads(0)

    return merge


def kernel(x, query, metric, token_size, r):
    del r
    n, t, c = x.shape
    qlast = query[:, -1:, :]
    mn = metric / jnp.linalg.norm(metric, axis=-1, keepdims=True)
    qn = qlast / jnp.linalg.norm(qlast, axis=-1, keepdims=True)
    ts_c = token_size
    ts_r = jnp.swapaxes(token_size, 1, 2)

    zeros = jnp.zeros((8, c), jnp.float32)
    coef, targ = _match(mn, qn, ts_c, ts_r)
    xflat = x.reshape(n * t, c)
    out = _make_merge(n, c)(xflat, coef.reshape(n * t),
                            targ.reshape(n * t), zeros)
    return out.reshape(n, _DST, c)
